# baseline (speedup 1.0000x reference)
import functools

import jax
import jax.numpy as jnp
from jax import lax
from jax.experimental import pallas as pl
from jax.experimental.pallas import tpu as pltpu
from jax.experimental.pallas import tpu_sc as plsc

_NC = 2
_NS = 16
_L = 16
_K = 80
_ZROWS = 40
_NWB = 10


def _sum_body(x_hbm, lg2_hbm, choff_hbm, so_hbm, si_hbm, co_hbm, ci_hbm,
              acco, acci, xbuf0, xbuf1, pb0, pb1, isb0, isb1, idb0, idb1,
              offbuf, hist, zbuf, sem0, sem1):
    c = lax.axis_index("c")
    s = lax.axis_index("s")
    E = x_hbm.shape[0]
    d = x_hbm.shape[1]
    dh = d // _NC
    e_per = E // _NS
    n_iter = e_per // _K
    n_nodes = hist.shape[0]
    n_wb = n_nodes // _NWB

    z16 = jnp.zeros((_L,), jnp.float32)
    o16 = jnp.ones((_L,), jnp.float32)

    def zrow(r, carry):
        def zcol(j, carry2):
            zbuf[r, pl.ds(j * _L, _L)] = z16
            return carry2
        return lax.fori_loop(0, dh // _L, zcol, carry)
    lax.fori_loop(0, _ZROWS, zrow, 0)

    def hrow(r, carry):
        hist[pl.ds(r * _L, _L)] = z16
        return carry
    lax.fori_loop(0, n_nodes // _L, hrow, 0)

    base_n = s * n_wb
    @pl.when(s < _NWB)
    def _():
        def zacc(i, carry):
            pltpu.sync_copy(zbuf, acco.at[pl.ds(base_n + i * _ZROWS, _ZROWS)])
            pltpu.sync_copy(zbuf, acci.at[pl.ds(base_n + i * _ZROWS, _ZROWS)])
            return carry
        lax.fori_loop(0, n_wb // _ZROWS, zacc, 0)

    plsc.subcore_barrier()

    xb = s * e_per

    def xcp(i, buf, sem):
        return pltpu.make_async_copy(
            x_hbm.at[pl.ds(xb + i * _K, _K), pl.ds(c * dh, dh)], buf, sem)

    def pcp(i, buf, sem):
        return pltpu.make_async_copy(
            lg2_hbm.at[pl.ds(2 * (xb + i * _K), 2 * _K)], buf, sem)

    def count(ib):
        def q(qi, carry):
            iv = ib[pl.ds(qi * _L, _L)]
            plsc.addupdate_scatter(hist, [iv], o16)
            return carry
        lax.fori_loop(0, _K // _L, q, 0)

    pltpu.sync_copy(choff_hbm.at[s], offbuf)
    lane = lax.iota(jnp.int32, _L)

    def deint(i, pb, sb, db):
        off = plsc.load_gather(offbuf, [jnp.zeros((_L,), jnp.int32) + i])
        for q in range(_K // _L):
            base = 2 * _L * q
            ivs = plsc.load_gather(pb, [lane * 2 + base])
            ivd = plsc.load_gather(pb, [lane * 2 + (base + 1)])
            sb[pl.ds(q * _L, _L)] = ivs + off
            db[pl.ds(q * _L, _L)] = ivd + off

    xcp(0, xbuf0, sem0).start()
    pcp(0, pb0, sem0).start()

    def step(j, carry):
        i0 = 2 * j
        i1 = i0 + 1
        xcp(i1, xbuf1, sem1).start()
        pcp(i1, pb1, sem1).start()
        xcp(i0, xbuf0, sem0).wait()
        pcp(i0, pb0, sem0).wait()
        deint(i0, pb0, isb0, idb0)
        pltpu.sync_copy(xbuf0, acco.at[isb0], add=True)
        pltpu.sync_copy(xbuf0, acci.at[idb0], add=True)

        @pl.when(c == 0)
        def _():
            count(isb0)

        @pl.when(c == 1)
        def _():
            count(idb0)

        @pl.when(j < n_iter // 2 - 1)
        def _():
            xcp(i0 + 2, xbuf0, sem0).start()
            pcp(i0 + 2, pb0, sem0).start()

        xcp(i1, xbuf1, sem1).wait()
        pcp(i1, pb1, sem1).wait()
        deint(i1, pb1, isb1, idb1)
        pltpu.sync_copy(xbuf1, acco.at[isb1], add=True)
        pltpu.sync_copy(xbuf1, acci.at[idb1], add=True)

        @pl.when(c == 0)
        def _():
            count(isb1)

        @pl.when(c == 1)
        def _():
            count(idb1)
        return carry
    lax.fori_loop(0, n_iter // 2, step, 0)

    plsc.subcore_barrier()

    @pl.when(s < _NWB)
    def _():
        pltpu.sync_copy(acco.at[pl.ds(base_n, n_wb)],
                        so_hbm.at[c, pl.ds(base_n, n_wb)])
        pltpu.sync_copy(acci.at[pl.ds(base_n, n_wb)],
                        si_hbm.at[c, pl.ds(base_n, n_wb)])

    @pl.when(c == 0)
    def _():
        pltpu.sync_copy(hist, co_hbm.at[s])

    @pl.when(c == 1)
    def _():
        pltpu.sync_copy(hist, ci_hbm.at[s])


def _sc_segment_sums(x, lg2, choff, num_nodes):
    E, d = x.shape
    dh = d // _NC
    n_iter = E // _NS // _K
    mesh = plsc.VectorSubcoreMesh(core_axis_name="c", subcore_axis_name="s",
                                  num_cores=_NC, num_subcores=_NS)
    f = pl.kernel(
        _sum_body,
        out_type=[
            jax.ShapeDtypeStruct((_NC, num_nodes, dh), jnp.float32),
            jax.ShapeDtypeStruct((_NC, num_nodes, dh), jnp.float32),
            jax.ShapeDtypeStruct((_NS, num_nodes), jnp.float32),
            jax.ShapeDtypeStruct((_NS, num_nodes), jnp.float32),
        ],
        mesh=mesh,
        scratch_types=[
            pltpu.VMEM_SHARED((num_nodes, dh), jnp.float32),
            pltpu.VMEM_SHARED((num_nodes, dh), jnp.float32),
            pltpu.VMEM((_K, dh), jnp.float32),
            pltpu.VMEM((_K, dh), jnp.float32),
            pltpu.VMEM((2 * _K,), jnp.int32),
            pltpu.VMEM((2 * _K,), jnp.int32),
            pltpu.VMEM((_K,), jnp.int32),
            pltpu.VMEM((_K,), jnp.int32),
            pltpu.VMEM((_K,), jnp.int32),
            pltpu.VMEM((_K,), jnp.int32),
            pltpu.VMEM((256,), jnp.int32),
            pltpu.VMEM((num_nodes,), jnp.float32),
            pltpu.VMEM((_ZROWS, dh), jnp.float32),
            pltpu.SemaphoreType.DMA,
            pltpu.SemaphoreType.DMA,
        ],
        compiler_params=pltpu.CompilerParams(use_tc_tiling_on_sc=False,
                                             needs_layout_passes=False),
    )
    return f(x, lg2, choff)


def _combine_body(so_ref, si_ref, hco_ref, hci_ref, out_ref):
    hdim = 42
    sumo = jnp.concatenate([so_ref[0], so_ref[1]], axis=1)
    sumi = jnp.concatenate([si_ref[0], si_ref[1]], axis=1)
    cnto = jnp.maximum(jnp.sum(hco_ref[...], axis=0), 1.0)[:, None]
    cnti = jnp.maximum(jnp.sum(hci_ref[...], axis=0), 1.0)[:, None]
    mo = sumo / cnto
    mi = sumi / cnti
    col = lax.broadcasted_iota(jnp.int32, mo.shape, 1)
    out_ref[...] = jnp.where(col < hdim, (mi - mo) * 0.5,
                             jnp.where(col < 2 * hdim, mi, mo))


def _combine(so, si, hcnt_out, hcnt_in):
    _, n, dh = so.shape
    return pl.pallas_call(
        _combine_body,
        out_shape=jax.ShapeDtypeStruct((n, _NC * dh), jnp.float32),
    )(so, si, hcnt_out, hcnt_in)


def kernel(x, lg_node_idx, org_graph_size, ptr):
    E, d = x.shape
    B = org_graph_size.shape[0]
    num_nodes = B * 625
    ogs = org_graph_size.astype(jnp.int32)
    ptr32 = ptr.astype(jnp.int32)
    cstart = jnp.arange(E // _K, dtype=jnp.int32)[:, None] * _K
    choff = jnp.sum(jnp.where(cstart >= ptr32[None, 1:B], ogs[None, :B - 1], 0),
                    axis=1, dtype=jnp.int32)
    n_iter = E // _K // _NS
    choff = jnp.pad(choff.reshape(_NS, n_iter), ((0, 0), (0, 256 - n_iter)))
    lg2 = lg_node_idx.reshape(-1)

    so, si, hco, hci = _sc_segment_sums(x, lg2, choff, num_nodes)
    return _combine(so, si, hco, hci)


# speedup vs baseline: 1.7350x; 1.7350x over previous
import functools

import jax
import jax.numpy as jnp
from jax import lax
from jax.experimental import pallas as pl
from jax.experimental.pallas import tpu as pltpu
from jax.experimental.pallas import tpu_sc as plsc

_NC = 2
_NS = 16
_L = 16
_K = 80
_ZROWS = 40
_NWB = 10
_NPAD = 10112


def _sum_body(x_hbm, sidx_hbm, didx_hbm, so_hbm, si_hbm, co_hbm, ci_hbm,
              acco, acci, xbuf0, xbuf1, isb0, isb1, idb0, idb1, hist, zbuf,
              sem0, sem1):
    c = lax.axis_index("c")
    s = lax.axis_index("s")
    E = x_hbm.shape[0]
    d = x_hbm.shape[1]
    dh = d // _NC
    e_per = E // _NS
    n_iter = e_per // _K
    n_nodes = hist.shape[0]
    n_wb = n_nodes // _NWB

    z16 = jnp.zeros((_L,), jnp.float32)
    o16 = jnp.ones((_L,), jnp.float32)

    def zrow(r, carry):
        def zcol(j, carry2):
            zbuf[r, pl.ds(j * _L, _L)] = z16
            return carry2
        return lax.fori_loop(0, dh // _L, zcol, carry)
    lax.fori_loop(0, _ZROWS, zrow, 0)

    def hrow(r, carry):
        hist[pl.ds(r * _L, _L)] = z16
        return carry
    lax.fori_loop(0, n_nodes // _L, hrow, 0)

    base_n = s * n_wb
    @pl.when(s < _NWB)
    def _():
        def zacc(i, carry):
            pltpu.sync_copy(zbuf, acco.at[pl.ds(base_n + i * _ZROWS, _ZROWS)])
            pltpu.sync_copy(zbuf, acci.at[pl.ds(base_n + i * _ZROWS, _ZROWS)])
            return carry
        lax.fori_loop(0, n_wb // _ZROWS, zacc, 0)

    plsc.subcore_barrier()

    xb = s * e_per

    def xcp(i, buf, sem):
        return pltpu.make_async_copy(
            x_hbm.at[pl.ds(xb + i * _K, _K), pl.ds(c * dh, dh)], buf, sem)

    def scp(i, buf, sem):
        return pltpu.make_async_copy(sidx_hbm.at[pl.ds(xb + i * _K, _K)],
                                     buf, sem)

    def dcp(i, buf, sem):
        return pltpu.make_async_copy(didx_hbm.at[pl.ds(xb + i * _K, _K)],
                                     buf, sem)

    def count(ib):
        def q(qi, carry):
            iv = ib[pl.ds(qi * _L, _L)]
            plsc.addupdate_scatter(hist, [iv], o16)
            return carry
        lax.fori_loop(0, _K // _L, q, 0)

    xcp(0, xbuf0, sem0).start()
    scp(0, isb0, sem0).start()
    dcp(0, idb0, sem0).start()

    def step(j, carry):
        i0 = 2 * j
        i1 = i0 + 1
        xcp(i1, xbuf1, sem1).start()
        scp(i1, isb1, sem1).start()
        dcp(i1, idb1, sem1).start()
        xcp(i0, xbuf0, sem0).wait()
        scp(i0, isb0, sem0).wait()
        dcp(i0, idb0, sem0).wait()
        pltpu.sync_copy(xbuf0, acco.at[isb0], add=True)
        pltpu.sync_copy(xbuf0, acci.at[idb0], add=True)

        @pl.when(c == 0)
        def _():
            count(isb0)

        @pl.when(c == 1)
        def _():
            count(idb0)

        @pl.when(j < n_iter // 2 - 1)
        def _():
            xcp(i0 + 2, xbuf0, sem0).start()
            scp(i0 + 2, isb0, sem0).start()
            dcp(i0 + 2, idb0, sem0).start()

        xcp(i1, xbuf1, sem1).wait()
        scp(i1, isb1, sem1).wait()
        dcp(i1, idb1, sem1).wait()
        pltpu.sync_copy(xbuf1, acco.at[isb1], add=True)
        pltpu.sync_copy(xbuf1, acci.at[idb1], add=True)

        @pl.when(c == 0)
        def _():
            count(isb1)

        @pl.when(c == 1)
        def _():
            count(idb1)
        return carry
    lax.fori_loop(0, n_iter // 2, step, 0)

    plsc.subcore_barrier()

    @pl.when(s < _NWB)
    def _():
        pltpu.sync_copy(acco.at[pl.ds(base_n, n_wb)],
                        so_hbm.at[pl.ds(base_n, n_wb), pl.ds(c * dh, dh)])
        pltpu.sync_copy(acci.at[pl.ds(base_n, n_wb)],
                        si_hbm.at[pl.ds(base_n, n_wb), pl.ds(c * dh, dh)])

    @pl.when(c == 0)
    def _():
        pltpu.sync_copy(hist, co_hbm.at[s, pl.ds(0, n_nodes)])

    @pl.when(c == 1)
    def _():
        pltpu.sync_copy(hist, ci_hbm.at[s, pl.ds(0, n_nodes)])


def _sc_segment_sums(x, sidx, didx, num_nodes):
    E, d = x.shape
    dh = d // _NC
    mesh = plsc.VectorSubcoreMesh(core_axis_name="c", subcore_axis_name="s",
                                  num_cores=_NC, num_subcores=_NS)
    f = pl.kernel(
        _sum_body,
        out_type=[
            jax.ShapeDtypeStruct((num_nodes, d), jnp.float32),
            jax.ShapeDtypeStruct((num_nodes, d), jnp.float32),
            jax.ShapeDtypeStruct((_NS, _NPAD), jnp.float32),
            jax.ShapeDtypeStruct((_NS, _NPAD), jnp.float32),
        ],
        mesh=mesh,
        scratch_types=[
            pltpu.VMEM_SHARED((num_nodes, dh), jnp.float32),
            pltpu.VMEM_SHARED((num_nodes, dh), jnp.float32),
            pltpu.VMEM((_K, dh), jnp.float32),
            pltpu.VMEM((_K, dh), jnp.float32),
            pltpu.VMEM((_K,), jnp.int32),
            pltpu.VMEM((_K,), jnp.int32),
            pltpu.VMEM((_K,), jnp.int32),
            pltpu.VMEM((_K,), jnp.int32),
            pltpu.VMEM((num_nodes,), jnp.float32),
            pltpu.VMEM((_ZROWS, dh), jnp.float32),
            pltpu.SemaphoreType.DMA,
            pltpu.SemaphoreType.DMA,
        ],
        compiler_params=pltpu.CompilerParams(use_tc_tiling_on_sc=False,
                                             needs_layout_passes=False),
    )
    return f(x, sidx, didx)


_PR = 250
_PC = 1280
_PBC = 128


def _prep_body(slg_ref, dlg_ref, ptr_ref, ogs_ref, sidx_ref, didx_ref):
    i = pl.program_id(0)
    B = ogs_ref.shape[0]
    r = lax.broadcasted_iota(jnp.int32, (_PR, _PBC), 0)
    col = lax.broadcasted_iota(jnp.int32, (_PR, _PBC), 1)
    e = r * _PC + i * _PBC + col
    pad = jnp.zeros((_PR, _PBC), jnp.int32)
    for h in range(B - 1):
        pad = pad + jnp.where(e >= ptr_ref[h + 1], ogs_ref[h], 0)
    sidx_ref[...] = slg_ref[...] + pad
    didx_ref[...] = dlg_ref[...] + pad


def _prep(s_lg, d_lg, ptr, ogs):
    grid = _PC // _PBC
    out = pl.pallas_call(
        _prep_body,
        grid=(grid,),
        in_specs=[
            pl.BlockSpec((_PR, _PBC), lambda i: (0, i)),
            pl.BlockSpec((_PR, _PBC), lambda i: (0, i)),
            pl.BlockSpec(memory_space=pltpu.SMEM),
            pl.BlockSpec(memory_space=pltpu.SMEM),
        ],
        out_specs=[
            pl.BlockSpec((_PR, _PBC), lambda i: (0, i)),
            pl.BlockSpec((_PR, _PBC), lambda i: (0, i)),
        ],
        out_shape=[
            jax.ShapeDtypeStruct((_PR, _PC), jnp.int32),
            jax.ShapeDtypeStruct((_PR, _PC), jnp.int32),
        ],
    )(s_lg.reshape(_PR, _PC), d_lg.reshape(_PR, _PC), ptr, ogs)
    return out[0].reshape(-1), out[1].reshape(-1)


def _combine_body(so_ref, si_ref, hco_ref, hci_ref, out_ref):
    hdim = 42
    n = so_ref.shape[0]
    cnto = jnp.maximum(jnp.sum(hco_ref[:, :n], axis=0), 1.0)[:, None]
    cnti = jnp.maximum(jnp.sum(hci_ref[:, :n], axis=0), 1.0)[:, None]
    mo = so_ref[...] / cnto
    mi = si_ref[...] / cnti
    col = lax.broadcasted_iota(jnp.int32, mo.shape, 1)
    out_ref[...] = jnp.where(col < hdim, (mi - mo) * 0.5,
                             jnp.where(col < 2 * hdim, mi, mo))


def _combine(so, si, hcnt_out, hcnt_in):
    n, d = so.shape
    return pl.pallas_call(
        _combine_body,
        out_shape=jax.ShapeDtypeStruct((n, d), jnp.float32),
    )(so, si, hcnt_out, hcnt_in)


def kernel(x, lg_node_idx, org_graph_size, ptr):
    E, d = x.shape
    B = org_graph_size.shape[0]
    num_nodes = B * 625
    ogs = org_graph_size.astype(jnp.int32)
    sidx, didx = _prep(lg_node_idx[:, 0], lg_node_idx[:, 1],
                       ptr.astype(jnp.int32), ogs)

    so, si, hco, hci = _sc_segment_sums(x, sidx, didx, num_nodes)
    return _combine(so, si, hco, hci)


# speedup vs baseline: 1.7353x; 1.0002x over previous
import jax
import jax.numpy as jnp
from jax import lax
from jax.experimental import pallas as pl
from jax.experimental.pallas import tpu as pltpu
from jax.experimental.pallas import tpu_sc as plsc

_NC = 2
_NS = 16
_L = 16
_K = 80
_ZROWS = 40
_NWB = 10
_NPAD = 10112


def _sum_body(x_hbm, sidx_hbm, didx_hbm, so_hbm, si_hbm, co_hbm, ci_hbm,
              acco, acci, xbuf0, xbuf1, isb0, isb1, idb0, idb1, hist, zbuf,
              sem0, sem1):
    c = lax.axis_index("c")
    s = lax.axis_index("s")
    E = x_hbm.shape[0]
    d = x_hbm.shape[1]
    dh = d // _NC
    e_per = E // _NS
    n_iter = e_per // _K
    n_nodes = hist.shape[0]
    n_wb = n_nodes // _NWB

    z16 = jnp.zeros((_L,), jnp.float32)
    o16 = jnp.ones((_L,), jnp.float32)

    def zrow(r, carry):
        def zcol(j, carry2):
            zbuf[r, pl.ds(j * _L, _L)] = z16
            return carry2
        return lax.fori_loop(0, dh // _L, zcol, carry)
    lax.fori_loop(0, _ZROWS, zrow, 0)

    def hrow(r, carry):
        hist[pl.ds(r * _L, _L)] = z16
        return carry
    lax.fori_loop(0, n_nodes // _L, hrow, 0)

    base_n = s * n_wb
    @pl.when(s < _NWB)
    def _():
        def zacc(i, carry):
            pltpu.sync_copy(zbuf, acco.at[pl.ds(base_n + i * _ZROWS, _ZROWS)])
            pltpu.sync_copy(zbuf, acci.at[pl.ds(base_n + i * _ZROWS, _ZROWS)])
            return carry
        lax.fori_loop(0, n_wb // _ZROWS, zacc, 0)

    plsc.subcore_barrier()

    xb = s * e_per

    def xcp(i, buf, sem):
        return pltpu.make_async_copy(
            x_hbm.at[pl.ds(xb + i * _K, _K), pl.ds(c * dh, dh)], buf, sem)

    def scp(i, buf, sem):
        return pltpu.make_async_copy(sidx_hbm.at[pl.ds(xb + i * _K, _K)],
                                     buf, sem)

    def dcp(i, buf, sem):
        return pltpu.make_async_copy(didx_hbm.at[pl.ds(xb + i * _K, _K)],
                                     buf, sem)

    def count(ib):
        def q(qi, carry):
            iv = ib[pl.ds(qi * _L, _L)]
            plsc.addupdate_scatter(hist, [iv], o16)
            return carry
        lax.fori_loop(0, _K // _L, q, 0)

    xcp(0, xbuf0, sem0).start()
    scp(0, isb0, sem0).start()
    dcp(0, idb0, sem0).start()

    def step(j, carry):
        i0 = 2 * j
        i1 = i0 + 1
        xcp(i1, xbuf1, sem1).start()
        scp(i1, isb1, sem1).start()
        dcp(i1, idb1, sem1).start()
        xcp(i0, xbuf0, sem0).wait()
        scp(i0, isb0, sem0).wait()
        dcp(i0, idb0, sem0).wait()
        pltpu.sync_copy(xbuf0, acco.at[isb0], add=True)
        pltpu.sync_copy(xbuf0, acci.at[idb0], add=True)

        @pl.when(c == 0)
        def _():
            count(isb0)

        @pl.when(c == 1)
        def _():
            count(idb0)

        @pl.when(j < n_iter // 2 - 1)
        def _():
            xcp(i0 + 2, xbuf0, sem0).start()
            scp(i0 + 2, isb0, sem0).start()
            dcp(i0 + 2, idb0, sem0).start()

        xcp(i1, xbuf1, sem1).wait()
        scp(i1, isb1, sem1).wait()
        dcp(i1, idb1, sem1).wait()
        pltpu.sync_copy(xbuf1, acco.at[isb1], add=True)
        pltpu.sync_copy(xbuf1, acci.at[idb1], add=True)

        @pl.when(c == 0)
        def _():
            count(isb1)

        @pl.when(c == 1)
        def _():
            count(idb1)
        return carry
    lax.fori_loop(0, n_iter // 2, step, 0)

    plsc.subcore_barrier()

    @pl.when(s < _NWB)
    def _():
        pltpu.sync_copy(acco.at[pl.ds(base_n, n_wb)],
                        so_hbm.at[pl.ds(base_n, n_wb), pl.ds(c * dh, dh)])
        pltpu.sync_copy(acci.at[pl.ds(base_n, n_wb)],
                        si_hbm.at[pl.ds(base_n, n_wb), pl.ds(c * dh, dh)])

    @pl.when(c == 0)
    def _():
        pltpu.sync_copy(hist, co_hbm.at[s, pl.ds(0, n_nodes)])

    @pl.when(c == 1)
    def _():
        pltpu.sync_copy(hist, ci_hbm.at[s, pl.ds(0, n_nodes)])


def _sc_segment_sums(x, sidx, didx, num_nodes):
    E, d = x.shape
    dh = d // _NC
    mesh = plsc.VectorSubcoreMesh(core_axis_name="c", subcore_axis_name="s",
                                  num_cores=_NC, num_subcores=_NS)
    f = pl.kernel(
        _sum_body,
        out_type=[
            jax.ShapeDtypeStruct((num_nodes, d), jnp.float32),
            jax.ShapeDtypeStruct((num_nodes, d), jnp.float32),
            jax.ShapeDtypeStruct((_NS, _NPAD), jnp.float32),
            jax.ShapeDtypeStruct((_NS, _NPAD), jnp.float32),
        ],
        mesh=mesh,
        scratch_types=[
            pltpu.VMEM_SHARED((num_nodes, dh), jnp.float32),
            pltpu.VMEM_SHARED((num_nodes, dh), jnp.float32),
            pltpu.VMEM((_K, dh), jnp.float32),
            pltpu.VMEM((_K, dh), jnp.float32),
            pltpu.VMEM((_K,), jnp.int32),
            pltpu.VMEM((_K,), jnp.int32),
            pltpu.VMEM((_K,), jnp.int32),
            pltpu.VMEM((_K,), jnp.int32),
            pltpu.VMEM((num_nodes,), jnp.float32),
            pltpu.VMEM((_ZROWS, dh), jnp.float32),
            pltpu.SemaphoreType.DMA,
            pltpu.SemaphoreType.DMA,
        ],
        compiler_params=pltpu.CompilerParams(use_tc_tiling_on_sc=False,
                                             needs_layout_passes=False),
    )
    return f(x, sidx, didx)


_PR = 250
_PC = 1280
_PBC = 128


def _prep_body(slg_ref, dlg_ref, ptr_ref, ogs_ref, sidx_ref, didx_ref):
    i = pl.program_id(0)
    B = ogs_ref.shape[0]
    r = lax.broadcasted_iota(jnp.int32, (_PR, _PBC), 0)
    col = lax.broadcasted_iota(jnp.int32, (_PR, _PBC), 1)
    e = r * _PC + i * _PBC + col
    pad = jnp.zeros((_PR, _PBC), jnp.int32)
    for h in range(B - 1):
        pad = pad + jnp.where(e >= ptr_ref[h + 1], ogs_ref[h], 0)
    sidx_ref[...] = slg_ref[...] + pad
    didx_ref[...] = dlg_ref[...] + pad


def _prep(s_lg, d_lg, ptr, ogs):
    grid = _PC // _PBC
    out = pl.pallas_call(
        _prep_body,
        grid=(grid,),
        in_specs=[
            pl.BlockSpec((_PR, _PBC), lambda i: (0, i)),
            pl.BlockSpec((_PR, _PBC), lambda i: (0, i)),
            pl.BlockSpec(memory_space=pltpu.SMEM),
            pl.BlockSpec(memory_space=pltpu.SMEM),
        ],
        out_specs=[
            pl.BlockSpec((_PR, _PBC), lambda i: (0, i)),
            pl.BlockSpec((_PR, _PBC), lambda i: (0, i)),
        ],
        out_shape=[
            jax.ShapeDtypeStruct((_PR, _PC), jnp.int32),
            jax.ShapeDtypeStruct((_PR, _PC), jnp.int32),
        ],
    )(s_lg.reshape(_PR, _PC), d_lg.reshape(_PR, _PC), ptr, ogs)
    return out[0].reshape(-1), out[1].reshape(-1)


def _combine_body(so_ref, si_ref, hco_ref, hci_ref, out_ref):
    hdim = 42
    n = so_ref.shape[0]
    cnto = jnp.maximum(jnp.sum(hco_ref[:, :n], axis=0), 1.0)[:, None]
    cnti = jnp.maximum(jnp.sum(hci_ref[:, :n], axis=0), 1.0)[:, None]
    mo = so_ref[...] / cnto
    mi = si_ref[...] / cnti
    col = lax.broadcasted_iota(jnp.int32, mo.shape, 1)
    out_ref[...] = jnp.where(col < hdim, (mi - mo) * 0.5,
                             jnp.where(col < 2 * hdim, mi, mo))


def _combine(so, si, hcnt_out, hcnt_in):
    n, d = so.shape
    return pl.pallas_call(
        _combine_body,
        out_shape=jax.ShapeDtypeStruct((n, d), jnp.float32),
    )(so, si, hcnt_out, hcnt_in)


def kernel(x, lg_node_idx, org_graph_size, ptr):
    E, d = x.shape
    B = org_graph_size.shape[0]
    num_nodes = B * 625
    ogs = org_graph_size.astype(jnp.int32)
    sidx, didx = _prep(lg_node_idx[:, 0], lg_node_idx[:, 1],
                       ptr.astype(jnp.int32), ogs)

    so, si, hco, hci = _sc_segment_sums(x, sidx, didx, num_nodes)
    return _combine(so, si, hco, hci)
